# Initial kernel scaffold; baseline (speedup 1.0000x reference)
import jax
import jax.numpy as jnp
from jax.experimental import pallas as pl


def kernel(xyz, features, W0, gamma0, beta0, W1, gamma1, beta1, W2, gamma2, beta2):
    raise NotImplementedError("write your pallas kernel here")


# speedup vs baseline: 6.7233x; 6.7233x over previous
import functools

import jax
import jax.numpy as jnp
from jax import lax
from jax.experimental import pallas as pl
from jax.experimental.pallas import tpu as pltpu
from jax.experimental.pallas import tpu_sc as plsc

_B, _N, _C = 8, 4096, 64
_M, _K = 1024, 32
_P = _B * _M * _K

_MB = 256
_PB = 4096
_MBF = 128

_NW = 32
_RW = _P // _NW
_CH = 128
_NCH = _RW // _CH

_BIG = 1e30


def _knn_body(nxyz_ref, xyzt_ref, idx_ref):
    b = pl.program_id(0)
    nb = nxyz_ref[...]
    xb = xyzt_ref[0]
    s2 = jnp.sum(nb * nb, axis=1, keepdims=True)
    d2 = jnp.sum(xb * xb, axis=0, keepdims=True)
    cross = lax.dot_general(nb, xb, (((1,), (0,)), ((), ())),
                            preferred_element_type=jnp.float32)
    d = s2 + d2 - 2.0 * cross

    lane_n = lax.broadcasted_iota(jnp.int32, (_MB, _N), 1).astype(jnp.float32)
    lane_k = lax.broadcasted_iota(jnp.int32, (_MB, _K), 1).astype(jnp.float32)

    def body(k, carry):
        d, acc = carry
        m = jnp.min(d, axis=1, keepdims=True)
        mi = jnp.where(d <= m, lane_n, jnp.float32(_N))
        am = jnp.min(mi, axis=1, keepdims=True)
        d = jnp.where(mi <= am, _BIG, d)
        acc = jnp.where(lane_k == k.astype(jnp.float32),
                        jnp.broadcast_to(am, (_MB, _K)), acc)
        return d, acc

    _, acc = lax.fori_loop(0, _K, body, (d, jnp.zeros((_MB, _K), jnp.float32)))
    idx_ref[...] = (acc + (b * _N).astype(jnp.float32)).astype(jnp.int32)


def _knn_ids(nxyz_pad, xyzt_pad):
    grid = (_B, _M // _MB)
    return pl.pallas_call(
        _knn_body,
        grid=grid,
        in_specs=[
            pl.BlockSpec((_MB, 8), lambda b, m: (b * (_M // _MB) + m, 0)),
            pl.BlockSpec((1, 8, _N), lambda b, m: (b, 0, 0)),
        ],
        out_specs=pl.BlockSpec((_MB, _K), lambda b, m: (b * (_M // _MB) + m, 0)),
        out_shape=jax.ShapeDtypeStruct((_B * _M, _K), jnp.int32),
    )(nxyz_pad, xyzt_pad)


def _sc_gather(idx3, tab):
    mesh = plsc.VectorSubcoreMesh(core_axis_name="c", subcore_axis_name="s")

    @functools.partial(
        pl.kernel,
        mesh=mesh,
        out_type=jax.ShapeDtypeStruct((_NW, _NCH, _CH, 128), jnp.float32),
        scratch_types=[
            pltpu.VMEM((_NCH, _CH), jnp.int32),
            pltpu.VMEM((_CH, 128), jnp.float32),
            pltpu.SemaphoreType.DMA,
        ],
    )
    def k(idx_hbm, tab_hbm, out_hbm, idx_v, buf, sem):
        wid = lax.axis_index("s") * 2 + lax.axis_index("c")
        pltpu.sync_copy(idx_hbm.at[wid], idx_v)

        def step(j, _):
            pltpu.async_copy(tab_hbm.at[idx_v.at[j]], buf, sem).wait()
            pltpu.sync_copy(buf, out_hbm.at[wid, j])
            return 0

        lax.fori_loop(0, _NCH, step, 0)

    return k(idx3, tab)


def _l0_body(g_ref, nx_ref, wc_ref, wx_ref, h_ref, s_ref, q_ref):
    @pl.when(pl.program_id(0) == 0)
    def _():
        s_ref[...] = jnp.zeros_like(s_ref)
        q_ref[...] = jnp.zeros_like(q_ref)

    g = g_ref[...]
    nx = nx_ref[...]
    corr = lax.dot_general(nx, wx_ref[...], (((1,), (0,)), ((), ())),
                           preferred_element_type=jnp.float32)
    corrb = jnp.broadcast_to(corr[:, None, :],
                             (_PB // _K, _K, 64)).reshape(_PB, 64)
    h = lax.dot_general(g, wc_ref[...], (((1,), (0,)), ((), ())),
                        preferred_element_type=jnp.float32) - corrb
    h_ref[...] = h
    s_ref[...] += jnp.sum(h, axis=0, keepdims=True)
    q_ref[...] += jnp.sum(h * h, axis=0, keepdims=True)


def _layer0(g, nxyz16, wc, wx):
    grid = (_P // _PB,)
    return pl.pallas_call(
        _l0_body,
        grid=grid,
        in_specs=[
            pl.BlockSpec((_PB, 128), lambda i: (i, 0)),
            pl.BlockSpec((_PB // _K, 16), lambda i: (i, 0)),
            pl.BlockSpec((128, 64), lambda i: (0, 0)),
            pl.BlockSpec((16, 64), lambda i: (0, 0)),
        ],
        out_specs=[
            pl.BlockSpec((_PB, 64), lambda i: (i, 0)),
            pl.BlockSpec((1, 64), lambda i: (0, 0)),
            pl.BlockSpec((1, 64), lambda i: (0, 0)),
        ],
        out_shape=[
            jax.ShapeDtypeStruct((_P, 64), jnp.float32),
            jax.ShapeDtypeStruct((1, 64), jnp.float32),
            jax.ShapeDtypeStruct((1, 64), jnp.float32),
        ],
    )(g, nxyz16, wc, wx)


def _mid_body(x_ref, w_ref, sc_ref, sh_ref, h_ref, s_ref, q_ref):
    @pl.when(pl.program_id(0) == 0)
    def _():
        s_ref[...] = jnp.zeros_like(s_ref)
        q_ref[...] = jnp.zeros_like(q_ref)

    a = jnp.maximum(x_ref[...] * sc_ref[...] + sh_ref[...], 0.0)
    h = lax.dot_general(a, w_ref[...], (((1,), (0,)), ((), ())),
                        preferred_element_type=jnp.float32)
    h_ref[...] = h
    s_ref[...] += jnp.sum(h, axis=0, keepdims=True)
    q_ref[...] += jnp.sum(h * h, axis=0, keepdims=True)


def _mid_layer(x, w, scale, shift):
    ci, co = w.shape
    grid = (_P // _PB,)
    return pl.pallas_call(
        _mid_body,
        grid=grid,
        in_specs=[
            pl.BlockSpec((_PB, ci), lambda i: (i, 0)),
            pl.BlockSpec((ci, co), lambda i: (0, 0)),
            pl.BlockSpec((1, ci), lambda i: (0, 0)),
            pl.BlockSpec((1, ci), lambda i: (0, 0)),
        ],
        out_specs=[
            pl.BlockSpec((_PB, co), lambda i: (i, 0)),
            pl.BlockSpec((1, co), lambda i: (0, 0)),
            pl.BlockSpec((1, co), lambda i: (0, 0)),
        ],
        out_shape=[
            jax.ShapeDtypeStruct((_P, co), jnp.float32),
            jax.ShapeDtypeStruct((1, co), jnp.float32),
            jax.ShapeDtypeStruct((1, co), jnp.float32),
        ],
    )(x, w, scale, shift)


def _pool_body(h_ref, sc_ref, sh_ref, o_ref):
    a = jnp.maximum(h_ref[...] * sc_ref[...] + sh_ref[...], 0.0)
    a = a.reshape(_MBF, _K, 256)
    m = jnp.max(a, axis=1)
    o_ref[0] = m.T


def _pool(h2, scale, shift):
    grid = (_B, _M // _MBF)
    return pl.pallas_call(
        _pool_body,
        grid=grid,
        in_specs=[
            pl.BlockSpec((_MBF * _K, 256),
                         lambda b, m: (b * (_M // _MBF) + m, 0)),
            pl.BlockSpec((1, 256), lambda b, m: (0, 0)),
            pl.BlockSpec((1, 256), lambda b, m: (0, 0)),
        ],
        out_specs=pl.BlockSpec((1, 256, _MBF), lambda b, m: (b, 0, m)),
        out_shape=jax.ShapeDtypeStruct((_B, 256, _M), jnp.float32),
    )(h2, scale, shift)


def _stats_to_affine(s, q, gamma, beta):
    mean = s[0] / _P
    var = q[0] / _P - mean * mean
    scale = gamma / jnp.sqrt(var + 1e-5)
    shift = beta - mean * scale
    return scale[None, :], shift[None, :]


def kernel(xyz, features, W0, gamma0, beta0, W1, gamma1, beta1, W2, gamma2, beta2):
    perm = jax.random.permutation(jax.random.key(1), _N)[:_M]
    new_xyz = jnp.take(xyz, perm, axis=1)

    nxyz_pad = jnp.pad(new_xyz, ((0, 0), (0, 0), (0, 5))).reshape(_B * _M, 8)
    xyzt_pad = jnp.pad(jnp.transpose(xyz, (0, 2, 1)),
                       ((0, 0), (0, 5), (0, 0)))

    idx = _knn_ids(nxyz_pad, xyzt_pad)
    idx3 = idx.reshape(_NW, _NCH, _CH)

    tab = jnp.concatenate(
        [jnp.transpose(features, (0, 2, 1)),
         jnp.pad(xyz, ((0, 0), (0, 0), (0, 61)))], axis=2).reshape(_B * _N, 128)

    g = _sc_gather(idx3, tab).reshape(_P, 128)

    nxyz16 = jnp.pad(new_xyz, ((0, 0), (0, 0), (0, 13))).reshape(_B * _M, 16)

    w0c = jnp.pad(jnp.transpose(W0), ((0, 61), (0, 0)))
    w0x = jnp.pad(jnp.transpose(W0[:, _C:]), ((0, 13), (0, 0)))

    h0, s0, q0 = _layer0(g, nxyz16, w0c, w0x)
    sc0, sh0 = _stats_to_affine(s0, q0, gamma0, beta0)

    h1, s1, q1 = _mid_layer(h0, jnp.transpose(W1), sc0, sh0)
    sc1, sh1 = _stats_to_affine(s1, q1, gamma1, beta1)

    h2, s2, q2 = _mid_layer(h1, jnp.transpose(W2), sc1, sh1)
    sc2, sh2 = _stats_to_affine(s2, q2, gamma2, beta2)

    new_features = _pool(h2, sc2, sh2)
    return new_xyz, new_features


# speedup vs baseline: 8.1743x; 1.2158x over previous
import functools

import jax
import jax.numpy as jnp
from jax import lax
from jax.experimental import pallas as pl
from jax.experimental.pallas import tpu as pltpu
from jax.experimental.pallas import tpu_sc as plsc

_B, _N, _C = 8, 4096, 64
_M, _K = 1024, 32
_P = _B * _M * _K

_MB = 256
_PB = 4096
_MBF = 128

_NW = 32
_RW = _P // _NW
_CH = 128
_NCH = _RW // _CH

_BIG = 1e30


def _knn_body(nxyz_ref, xyzt_ref, idx_ref, d_ref, acc_ref):
    b = pl.program_id(0)
    nb = nxyz_ref[...]
    xb = xyzt_ref[0]
    s2 = jnp.sum(nb * nb, axis=1, keepdims=True)
    d2 = jnp.sum(xb * xb, axis=0, keepdims=True)
    cross = lax.dot_general(nb, xb, (((1,), (0,)), ((), ())),
                            preferred_element_type=jnp.float32)
    d_ref[...] = s2 + d2 - 2.0 * cross

    lane_n = lax.broadcasted_iota(jnp.int32, (_MB, _N), 1).astype(jnp.float32)
    lane_k = lax.broadcasted_iota(jnp.int32, (_MB, _K), 1).astype(jnp.float32)

    def body(k, _):
        d = d_ref[...]
        m = jnp.min(d, axis=1, keepdims=True)
        am = jnp.min(jnp.where(d <= m, lane_n, jnp.float32(_N)),
                     axis=1, keepdims=True)
        d_ref[...] = jnp.where((d <= m) & (lane_n <= am), _BIG, d)
        acc_ref[...] = jnp.where(lane_k == k.astype(jnp.float32),
                                 jnp.broadcast_to(am, (_MB, _K)), acc_ref[...])
        return 0

    lax.fori_loop(0, _K, body, 0)
    idx_ref[...] = (acc_ref[...] + (b * _N).astype(jnp.float32)).astype(jnp.int32)


def _knn_ids(nxyz_pad, xyzt_pad):
    grid = (_B, _M // _MB)
    return pl.pallas_call(
        _knn_body,
        grid=grid,
        in_specs=[
            pl.BlockSpec((_MB, 8), lambda b, m: (b * (_M // _MB) + m, 0)),
            pl.BlockSpec((1, 8, _N), lambda b, m: (b, 0, 0)),
        ],
        out_specs=pl.BlockSpec((_MB, _K), lambda b, m: (b * (_M // _MB) + m, 0)),
        out_shape=jax.ShapeDtypeStruct((_B * _M, _K), jnp.int32),
        scratch_shapes=[pltpu.VMEM((_MB, _N), jnp.float32),
                        pltpu.VMEM((_MB, _K), jnp.float32)],
    )(nxyz_pad, xyzt_pad)


def _sc_gather(idx3, tab):
    mesh = plsc.VectorSubcoreMesh(core_axis_name="c", subcore_axis_name="s")

    @functools.partial(
        pl.kernel,
        mesh=mesh,
        out_type=jax.ShapeDtypeStruct((_NW, _NCH, _CH, 128), jnp.float32),
        scratch_types=[
            pltpu.VMEM((_NCH, _CH), jnp.int32),
            pltpu.VMEM((_CH, 128), jnp.float32),
            pltpu.SemaphoreType.DMA,
        ],
    )
    def k(idx_hbm, tab_hbm, out_hbm, idx_v, buf, sem):
        wid = lax.axis_index("s") * 2 + lax.axis_index("c")
        pltpu.sync_copy(idx_hbm.at[wid], idx_v)

        def step(j, _):
            pltpu.async_copy(tab_hbm.at[idx_v.at[j]], buf, sem).wait()
            pltpu.sync_copy(buf, out_hbm.at[wid, j])
            return 0

        lax.fori_loop(0, _NCH, step, 0)

    return k(idx3, tab)


def _l0_body(g_ref, nx_ref, wc_ref, wx_ref, h_ref, s_ref, q_ref):
    @pl.when(pl.program_id(0) == 0)
    def _():
        s_ref[...] = jnp.zeros_like(s_ref)
        q_ref[...] = jnp.zeros_like(q_ref)

    g = g_ref[...]
    nx = nx_ref[...]
    corr = lax.dot_general(nx, wx_ref[...], (((1,), (0,)), ((), ())),
                           preferred_element_type=jnp.float32)
    corrb = jnp.broadcast_to(corr[:, None, :],
                             (_PB // _K, _K, 64)).reshape(_PB, 64)
    h = lax.dot_general(g, wc_ref[...], (((1,), (0,)), ((), ())),
                        preferred_element_type=jnp.float32) - corrb
    h_ref[...] = h
    s_ref[...] += jnp.sum(h, axis=0, keepdims=True)
    q_ref[...] += jnp.sum(h * h, axis=0, keepdims=True)


def _layer0(g, nxyz16, wc, wx):
    grid = (_P // _PB,)
    return pl.pallas_call(
        _l0_body,
        grid=grid,
        in_specs=[
            pl.BlockSpec((_PB, 128), lambda i: (i, 0)),
            pl.BlockSpec((_PB // _K, 16), lambda i: (i, 0)),
            pl.BlockSpec((128, 64), lambda i: (0, 0)),
            pl.BlockSpec((16, 64), lambda i: (0, 0)),
        ],
        out_specs=[
            pl.BlockSpec((_PB, 64), lambda i: (i, 0)),
            pl.BlockSpec((1, 64), lambda i: (0, 0)),
            pl.BlockSpec((1, 64), lambda i: (0, 0)),
        ],
        out_shape=[
            jax.ShapeDtypeStruct((_P, 64), jnp.float32),
            jax.ShapeDtypeStruct((1, 64), jnp.float32),
            jax.ShapeDtypeStruct((1, 64), jnp.float32),
        ],
    )(g, nxyz16, wc, wx)


def _mid_body(x_ref, w_ref, sc_ref, sh_ref, h_ref, s_ref, q_ref):
    @pl.when(pl.program_id(0) == 0)
    def _():
        s_ref[...] = jnp.zeros_like(s_ref)
        q_ref[...] = jnp.zeros_like(q_ref)

    a = jnp.maximum(x_ref[...] * sc_ref[...] + sh_ref[...], 0.0)
    h = lax.dot_general(a, w_ref[...], (((1,), (0,)), ((), ())),
                        preferred_element_type=jnp.float32)
    h_ref[...] = h
    s_ref[...] += jnp.sum(h, axis=0, keepdims=True)
    q_ref[...] += jnp.sum(h * h, axis=0, keepdims=True)


def _mid_layer(x, w, scale, shift):
    ci, co = w.shape
    grid = (_P // _PB,)
    return pl.pallas_call(
        _mid_body,
        grid=grid,
        in_specs=[
            pl.BlockSpec((_PB, ci), lambda i: (i, 0)),
            pl.BlockSpec((ci, co), lambda i: (0, 0)),
            pl.BlockSpec((1, ci), lambda i: (0, 0)),
            pl.BlockSpec((1, ci), lambda i: (0, 0)),
        ],
        out_specs=[
            pl.BlockSpec((_PB, co), lambda i: (i, 0)),
            pl.BlockSpec((1, co), lambda i: (0, 0)),
            pl.BlockSpec((1, co), lambda i: (0, 0)),
        ],
        out_shape=[
            jax.ShapeDtypeStruct((_P, co), jnp.float32),
            jax.ShapeDtypeStruct((1, co), jnp.float32),
            jax.ShapeDtypeStruct((1, co), jnp.float32),
        ],
    )(x, w, scale, shift)


def _pool_body(h_ref, sc_ref, sh_ref, o_ref):
    a = jnp.maximum(h_ref[...] * sc_ref[...] + sh_ref[...], 0.0)
    a = a.reshape(_MBF, _K, 256)
    m = jnp.max(a, axis=1)
    o_ref[0] = m.T


def _pool(h2, scale, shift):
    grid = (_B, _M // _MBF)
    return pl.pallas_call(
        _pool_body,
        grid=grid,
        in_specs=[
            pl.BlockSpec((_MBF * _K, 256),
                         lambda b, m: (b * (_M // _MBF) + m, 0)),
            pl.BlockSpec((1, 256), lambda b, m: (0, 0)),
            pl.BlockSpec((1, 256), lambda b, m: (0, 0)),
        ],
        out_specs=pl.BlockSpec((1, 256, _MBF), lambda b, m: (b, 0, m)),
        out_shape=jax.ShapeDtypeStruct((_B, 256, _M), jnp.float32),
    )(h2, scale, shift)


def _stats_to_affine(s, q, gamma, beta):
    mean = s[0] / _P
    var = q[0] / _P - mean * mean
    scale = gamma / jnp.sqrt(var + 1e-5)
    shift = beta - mean * scale
    return scale[None, :], shift[None, :]


def kernel(xyz, features, W0, gamma0, beta0, W1, gamma1, beta1, W2, gamma2, beta2):
    perm = jax.random.permutation(jax.random.key(1), _N)[:_M]
    new_xyz = jnp.take(xyz, perm, axis=1)

    nxyz_pad = jnp.pad(new_xyz, ((0, 0), (0, 0), (0, 5))).reshape(_B * _M, 8)
    xyzt_pad = jnp.pad(jnp.transpose(xyz, (0, 2, 1)),
                       ((0, 0), (0, 5), (0, 0)))

    idx = _knn_ids(nxyz_pad, xyzt_pad)
    idx3 = idx.reshape(_NW, _NCH, _CH)

    tab = jnp.concatenate(
        [jnp.transpose(features, (0, 2, 1)),
         jnp.pad(xyz, ((0, 0), (0, 0), (0, 61)))], axis=2).reshape(_B * _N, 128)

    g = _sc_gather(idx3, tab).reshape(_P, 128)

    nxyz16 = jnp.pad(new_xyz, ((0, 0), (0, 0), (0, 13))).reshape(_B * _M, 16)

    w0c = jnp.pad(jnp.transpose(W0), ((0, 61), (0, 0)))
    w0x = jnp.pad(jnp.transpose(W0[:, _C:]), ((0, 13), (0, 0)))

    h0, s0, q0 = _layer0(g, nxyz16, w0c, w0x)
    sc0, sh0 = _stats_to_affine(s0, q0, gamma0, beta0)

    h1, s1, q1 = _mid_layer(h0, jnp.transpose(W1), sc0, sh0)
    sc1, sh1 = _stats_to_affine(s1, q1, gamma1, beta1)

    h2, s2, q2 = _mid_layer(h1, jnp.transpose(W2), sc1, sh1)
    sc2, sh2 = _stats_to_affine(s2, q2, gamma2, beta2)

    new_features = _pool(h2, sc2, sh2)
    return new_xyz, new_features


# speedup vs baseline: 9.5932x; 1.1736x over previous
import functools

import jax
import jax.numpy as jnp
from jax import lax
from jax.experimental import pallas as pl
from jax.experimental.pallas import tpu as pltpu
from jax.experimental.pallas import tpu_sc as plsc

_B, _N, _C = 8, 4096, 64
_M, _K = 1024, 32
_P = _B * _M * _K

_MB = 256
_PB = 4096
_MBF = 128

_NW = 32
_RW = _P // _NW
_CH = 128
_NCH = _RW // _CH

_BIG = 1e30


def _knn_body(nxyz_ref, xyzt_ref, idx_ref, d_ref, acc_ref):
    b = pl.program_id(0)
    nb = nxyz_ref[...]
    xb = xyzt_ref[0]
    s2 = jnp.sum(nb * nb, axis=1, keepdims=True)
    d2 = jnp.sum(xb * xb, axis=0, keepdims=True)
    cross = lax.dot_general(nb, xb, (((1,), (0,)), ((), ())),
                            preferred_element_type=jnp.float32)
    d0 = s2 + d2 - 2.0 * cross
    d_ref[...] = d0
    m0 = jnp.min(d0, axis=1, keepdims=True)

    lane_n = lax.broadcasted_iota(jnp.int32, (_MB, _N), 1).astype(jnp.float32)
    lane_k = lax.broadcasted_iota(jnp.int32, (_MB, _K), 1).astype(jnp.float32)

    def body(k, m):
        d = d_ref[...]
        am = jnp.min(jnp.where(d <= m, lane_n, jnp.float32(_N)),
                     axis=1, keepdims=True)
        dn = jnp.where(lane_n == am, _BIG, d)
        d_ref[...] = dn
        acc_ref[...] = jnp.where(lane_k == k.astype(jnp.float32),
                                 jnp.broadcast_to(am, (_MB, _K)), acc_ref[...])
        return jnp.min(dn, axis=1, keepdims=True)

    lax.fori_loop(0, _K, body, m0)
    idx_ref[...] = (acc_ref[...] + (b * _N).astype(jnp.float32)).astype(jnp.int32)


def _knn_ids(nxyz_pad, xyzt_pad):
    grid = (_B, _M // _MB)
    return pl.pallas_call(
        _knn_body,
        grid=grid,
        in_specs=[
            pl.BlockSpec((_MB, 8), lambda b, m: (b * (_M // _MB) + m, 0)),
            pl.BlockSpec((1, 8, _N), lambda b, m: (b, 0, 0)),
        ],
        out_specs=pl.BlockSpec((_MB, _K), lambda b, m: (b * (_M // _MB) + m, 0)),
        out_shape=jax.ShapeDtypeStruct((_B * _M, _K), jnp.int32),
        scratch_shapes=[pltpu.VMEM((_MB, _N), jnp.float32),
                        pltpu.VMEM((_MB, _K), jnp.float32)],
    )(nxyz_pad, xyzt_pad)


def _sc_gather(idx3, tab):
    mesh = plsc.VectorSubcoreMesh(core_axis_name="c", subcore_axis_name="s")

    @functools.partial(
        pl.kernel,
        mesh=mesh,
        out_type=jax.ShapeDtypeStruct((_NW, _NCH, _CH, 128), jnp.float32),
        scratch_types=[
            pltpu.VMEM((_NCH, _CH), jnp.int32),
            pltpu.VMEM((_CH, 128), jnp.float32),
            pltpu.SemaphoreType.DMA,
        ],
    )
    def k(idx_hbm, tab_hbm, out_hbm, idx_v, buf, sem):
        wid = lax.axis_index("s") * 2 + lax.axis_index("c")
        pltpu.sync_copy(idx_hbm.at[wid], idx_v)

        def step(j, _):
            pltpu.async_copy(tab_hbm.at[idx_v.at[j]], buf, sem).wait()
            pltpu.sync_copy(buf, out_hbm.at[wid, j])
            return 0

        lax.fori_loop(0, _NCH, step, 0)

    return k(idx3, tab)


def _l0_body(g_ref, nx_ref, wc_ref, wx_ref, h_ref, s_ref, q_ref):
    @pl.when(pl.program_id(0) == 0)
    def _():
        s_ref[...] = jnp.zeros_like(s_ref)
        q_ref[...] = jnp.zeros_like(q_ref)

    g = g_ref[...]
    nx = nx_ref[...]
    corr = lax.dot_general(nx, wx_ref[...], (((1,), (0,)), ((), ())),
                           preferred_element_type=jnp.float32)
    corrb = jnp.broadcast_to(corr[:, None, :],
                             (_PB // _K, _K, 64)).reshape(_PB, 64)
    h = lax.dot_general(g, wc_ref[...], (((1,), (0,)), ((), ())),
                        preferred_element_type=jnp.float32) - corrb
    h_ref[...] = h
    s_ref[...] += jnp.sum(h, axis=0, keepdims=True)
    q_ref[...] += jnp.sum(h * h, axis=0, keepdims=True)


def _layer0(g, nxyz16, wc, wx):
    grid = (_P // _PB,)
    return pl.pallas_call(
        _l0_body,
        grid=grid,
        in_specs=[
            pl.BlockSpec((_PB, 128), lambda i: (i, 0)),
            pl.BlockSpec((_PB // _K, 16), lambda i: (i, 0)),
            pl.BlockSpec((128, 64), lambda i: (0, 0)),
            pl.BlockSpec((16, 64), lambda i: (0, 0)),
        ],
        out_specs=[
            pl.BlockSpec((_PB, 64), lambda i: (i, 0)),
            pl.BlockSpec((1, 64), lambda i: (0, 0)),
            pl.BlockSpec((1, 64), lambda i: (0, 0)),
        ],
        out_shape=[
            jax.ShapeDtypeStruct((_P, 64), jnp.float32),
            jax.ShapeDtypeStruct((1, 64), jnp.float32),
            jax.ShapeDtypeStruct((1, 64), jnp.float32),
        ],
    )(g, nxyz16, wc, wx)


def _mid_body(x_ref, w_ref, sc_ref, sh_ref, h_ref, s_ref, q_ref):
    @pl.when(pl.program_id(0) == 0)
    def _():
        s_ref[...] = jnp.zeros_like(s_ref)
        q_ref[...] = jnp.zeros_like(q_ref)

    a = jnp.maximum(x_ref[...] * sc_ref[...] + sh_ref[...], 0.0)
    h = lax.dot_general(a, w_ref[...], (((1,), (0,)), ((), ())),
                        preferred_element_type=jnp.float32)
    h_ref[...] = h
    s_ref[...] += jnp.sum(h, axis=0, keepdims=True)
    q_ref[...] += jnp.sum(h * h, axis=0, keepdims=True)


def _mid_layer(x, w, scale, shift):
    ci, co = w.shape
    grid = (_P // _PB,)
    return pl.pallas_call(
        _mid_body,
        grid=grid,
        in_specs=[
            pl.BlockSpec((_PB, ci), lambda i: (i, 0)),
            pl.BlockSpec((ci, co), lambda i: (0, 0)),
            pl.BlockSpec((1, ci), lambda i: (0, 0)),
            pl.BlockSpec((1, ci), lambda i: (0, 0)),
        ],
        out_specs=[
            pl.BlockSpec((_PB, co), lambda i: (i, 0)),
            pl.BlockSpec((1, co), lambda i: (0, 0)),
            pl.BlockSpec((1, co), lambda i: (0, 0)),
        ],
        out_shape=[
            jax.ShapeDtypeStruct((_P, co), jnp.float32),
            jax.ShapeDtypeStruct((1, co), jnp.float32),
            jax.ShapeDtypeStruct((1, co), jnp.float32),
        ],
    )(x, w, scale, shift)


def _pool_body(h_ref, sc_ref, sh_ref, o_ref):
    a = jnp.maximum(h_ref[...] * sc_ref[...] + sh_ref[...], 0.0)
    a = a.reshape(_MBF, _K, 256)
    m = jnp.max(a, axis=1)
    o_ref[0] = m.T


def _pool(h2, scale, shift):
    grid = (_B, _M // _MBF)
    return pl.pallas_call(
        _pool_body,
        grid=grid,
        in_specs=[
            pl.BlockSpec((_MBF * _K, 256),
                         lambda b, m: (b * (_M // _MBF) + m, 0)),
            pl.BlockSpec((1, 256), lambda b, m: (0, 0)),
            pl.BlockSpec((1, 256), lambda b, m: (0, 0)),
        ],
        out_specs=pl.BlockSpec((1, 256, _MBF), lambda b, m: (b, 0, m)),
        out_shape=jax.ShapeDtypeStruct((_B, 256, _M), jnp.float32),
    )(h2, scale, shift)


def _stats_to_affine(s, q, gamma, beta):
    mean = s[0] / _P
    var = q[0] / _P - mean * mean
    scale = gamma / jnp.sqrt(var + 1e-5)
    shift = beta - mean * scale
    return scale[None, :], shift[None, :]


def kernel(xyz, features, W0, gamma0, beta0, W1, gamma1, beta1, W2, gamma2, beta2):
    perm = jax.random.permutation(jax.random.key(1), _N)[:_M]
    new_xyz = jnp.take(xyz, perm, axis=1)

    nxyz_pad = jnp.pad(new_xyz, ((0, 0), (0, 0), (0, 5))).reshape(_B * _M, 8)
    xyzt_pad = jnp.pad(jnp.transpose(xyz, (0, 2, 1)),
                       ((0, 0), (0, 5), (0, 0)))

    idx = _knn_ids(nxyz_pad, xyzt_pad)
    idx3 = idx.reshape(_NW, _NCH, _CH)

    tab = jnp.concatenate(
        [jnp.transpose(features, (0, 2, 1)),
         jnp.pad(xyz, ((0, 0), (0, 0), (0, 61)))], axis=2).reshape(_B * _N, 128)

    g = _sc_gather(idx3, tab).reshape(_P, 128)

    nxyz16 = jnp.pad(new_xyz, ((0, 0), (0, 0), (0, 13))).reshape(_B * _M, 16)

    w0c = jnp.pad(jnp.transpose(W0), ((0, 61), (0, 0)))
    w0x = jnp.pad(jnp.transpose(W0[:, _C:]), ((0, 13), (0, 0)))

    h0, s0, q0 = _layer0(g, nxyz16, w0c, w0x)
    sc0, sh0 = _stats_to_affine(s0, q0, gamma0, beta0)

    h1, s1, q1 = _mid_layer(h0, jnp.transpose(W1), sc0, sh0)
    sc1, sh1 = _stats_to_affine(s1, q1, gamma1, beta1)

    h2, s2, q2 = _mid_layer(h1, jnp.transpose(W2), sc1, sh1)
    sc2, sh2 = _stats_to_affine(s2, q2, gamma2, beta2)

    new_features = _pool(h2, sc2, sh2)
    return new_xyz, new_features


# speedup vs baseline: 10.2061x; 1.0639x over previous
import functools

import jax
import jax.numpy as jnp
from jax import lax
from jax.experimental import pallas as pl
from jax.experimental.pallas import tpu as pltpu
from jax.experimental.pallas import tpu_sc as plsc

_B, _N, _C = 8, 4096, 64
_M, _K = 1024, 32
_P = _B * _M * _K

_MB = 256
_PB = 4096
_MBF = 128

_NW = 32
_RW = _P // _NW
_CH = 128
_NCH = _RW // _CH

_BIG = 1e30


def _knn_body(nxyz_ref, xyzt_ref, idx_ref, d_ref, acc_ref):
    b = pl.program_id(0)
    nb = nxyz_ref[...]
    xb = xyzt_ref[0]
    s2 = jnp.sum(nb * nb, axis=1, keepdims=True)
    d2 = jnp.sum(xb * xb, axis=0, keepdims=True)
    cross = lax.dot_general(nb, xb, (((1,), (0,)), ((), ())),
                            preferred_element_type=jnp.float32)
    d0 = s2 + d2 - 2.0 * cross
    d_ref[...] = d0
    m0 = jnp.min(d0, axis=1, keepdims=True)

    lane_n = lax.broadcasted_iota(jnp.int32, (_MB, _N), 1).astype(jnp.float32)
    lane_k = lax.broadcasted_iota(jnp.int32, (_MB, _K), 1).astype(jnp.float32)

    def body(k, m):
        d = d_ref[...]
        am = jnp.min(jnp.where(d <= m, lane_n, jnp.float32(_N)),
                     axis=1, keepdims=True)
        dn = jnp.where(lane_n == am, _BIG, d)
        d_ref[...] = dn
        acc_ref[...] = jnp.where(lane_k == k.astype(jnp.float32),
                                 jnp.broadcast_to(am, (_MB, _K)), acc_ref[...])
        return jnp.min(dn, axis=1, keepdims=True)

    lax.fori_loop(0, _K, body, m0)
    idx_ref[...] = (acc_ref[...] + (b * _N).astype(jnp.float32)).astype(jnp.int32)


def _knn_ids(nxyz_pad, xyzt_pad):
    grid = (_B, _M // _MB)
    return pl.pallas_call(
        _knn_body,
        grid=grid,
        in_specs=[
            pl.BlockSpec((_MB, 8), lambda b, m: (b * (_M // _MB) + m, 0)),
            pl.BlockSpec((1, 8, _N), lambda b, m: (b, 0, 0)),
        ],
        out_specs=pl.BlockSpec((_MB, _K), lambda b, m: (b * (_M // _MB) + m, 0)),
        out_shape=jax.ShapeDtypeStruct((_B * _M, _K), jnp.int32),
        scratch_shapes=[pltpu.VMEM((_MB, _N), jnp.float32),
                        pltpu.VMEM((_MB, _K), jnp.float32)],
    )(nxyz_pad, xyzt_pad)


def _sc_gather(idx3, tab):
    mesh = plsc.VectorSubcoreMesh(core_axis_name="c", subcore_axis_name="s")

    @functools.partial(
        pl.kernel,
        mesh=mesh,
        out_type=jax.ShapeDtypeStruct((_NW, _NCH, _CH, 128), jnp.float32),
        scratch_types=[
            pltpu.VMEM((_NCH, _CH), jnp.int32),
            pltpu.VMEM((_CH, 128), jnp.float32),
            pltpu.VMEM((_CH, 128), jnp.float32),
            pltpu.SemaphoreType.DMA,
            pltpu.SemaphoreType.DMA,
        ],
    )
    def k(idx_hbm, tab_hbm, out_hbm, idx_v, buf0, buf1, sem0, sem1):
        wid = lax.axis_index("s") * 2 + lax.axis_index("c")
        pltpu.sync_copy(idx_hbm.at[wid], idx_v)

        pltpu.async_copy(tab_hbm.at[idx_v.at[0]], buf0, sem0)

        def step(j2, _):
            j = j2 * 2
            pltpu.async_copy(tab_hbm.at[idx_v.at[j + 1]], buf1, sem1)
            pltpu.make_async_copy(tab_hbm.at[idx_v.at[j]], buf0, sem0).wait()
            pltpu.sync_copy(buf0, out_hbm.at[wid, j])

            @pl.when(j + 2 < _NCH)
            def _():
                pltpu.async_copy(tab_hbm.at[idx_v.at[j + 2]], buf0, sem0)

            pltpu.make_async_copy(tab_hbm.at[idx_v.at[j + 1]], buf1, sem1).wait()
            pltpu.sync_copy(buf1, out_hbm.at[wid, j + 1])
            return 0

        lax.fori_loop(0, _NCH // 2, step, 0)

    return k(idx3, tab)


def _l0_body(g_ref, nx_ref, wc_ref, wx_ref, h_ref, s_ref, q_ref):
    @pl.when(pl.program_id(0) == 0)
    def _():
        s_ref[...] = jnp.zeros_like(s_ref)
        q_ref[...] = jnp.zeros_like(q_ref)

    g = g_ref[...]
    nx = nx_ref[...]
    corr = lax.dot_general(nx, wx_ref[...], (((1,), (0,)), ((), ())),
                           preferred_element_type=jnp.float32)
    corrb = jnp.broadcast_to(corr[:, None, :],
                             (_PB // _K, _K, 64)).reshape(_PB, 64)
    h = lax.dot_general(g, wc_ref[...], (((1,), (0,)), ((), ())),
                        preferred_element_type=jnp.float32) - corrb
    h_ref[...] = h
    s_ref[...] += jnp.sum(h, axis=0, keepdims=True)
    q_ref[...] += jnp.sum(h * h, axis=0, keepdims=True)


def _layer0(g, nxyz16, wc, wx):
    grid = (_P // _PB,)
    return pl.pallas_call(
        _l0_body,
        grid=grid,
        in_specs=[
            pl.BlockSpec((_PB, 128), lambda i: (i, 0)),
            pl.BlockSpec((_PB // _K, 16), lambda i: (i, 0)),
            pl.BlockSpec((128, 64), lambda i: (0, 0)),
            pl.BlockSpec((16, 64), lambda i: (0, 0)),
        ],
        out_specs=[
            pl.BlockSpec((_PB, 64), lambda i: (i, 0)),
            pl.BlockSpec((1, 64), lambda i: (0, 0)),
            pl.BlockSpec((1, 64), lambda i: (0, 0)),
        ],
        out_shape=[
            jax.ShapeDtypeStruct((_P, 64), jnp.float32),
            jax.ShapeDtypeStruct((1, 64), jnp.float32),
            jax.ShapeDtypeStruct((1, 64), jnp.float32),
        ],
    )(g, nxyz16, wc, wx)


def _mid_body(x_ref, w_ref, sc_ref, sh_ref, h_ref, s_ref, q_ref):
    @pl.when(pl.program_id(0) == 0)
    def _():
        s_ref[...] = jnp.zeros_like(s_ref)
        q_ref[...] = jnp.zeros_like(q_ref)

    a = jnp.maximum(x_ref[...] * sc_ref[...] + sh_ref[...], 0.0)
    h = lax.dot_general(a, w_ref[...], (((1,), (0,)), ((), ())),
                        preferred_element_type=jnp.float32)
    h_ref[...] = h
    s_ref[...] += jnp.sum(h, axis=0, keepdims=True)
    q_ref[...] += jnp.sum(h * h, axis=0, keepdims=True)


def _mid_layer(x, w, scale, shift):
    ci, co = w.shape
    grid = (_P // _PB,)
    return pl.pallas_call(
        _mid_body,
        grid=grid,
        in_specs=[
            pl.BlockSpec((_PB, ci), lambda i: (i, 0)),
            pl.BlockSpec((ci, co), lambda i: (0, 0)),
            pl.BlockSpec((1, ci), lambda i: (0, 0)),
            pl.BlockSpec((1, ci), lambda i: (0, 0)),
        ],
        out_specs=[
            pl.BlockSpec((_PB, co), lambda i: (i, 0)),
            pl.BlockSpec((1, co), lambda i: (0, 0)),
            pl.BlockSpec((1, co), lambda i: (0, 0)),
        ],
        out_shape=[
            jax.ShapeDtypeStruct((_P, co), jnp.float32),
            jax.ShapeDtypeStruct((1, co), jnp.float32),
            jax.ShapeDtypeStruct((1, co), jnp.float32),
        ],
    )(x, w, scale, shift)


def _stats_body(x_ref, w_ref, sc_ref, sh_ref, s_ref, q_ref):
    @pl.when(pl.program_id(0) == 0)
    def _():
        s_ref[...] = jnp.zeros_like(s_ref)
        q_ref[...] = jnp.zeros_like(q_ref)

    a = jnp.maximum(x_ref[...] * sc_ref[...] + sh_ref[...], 0.0)
    h = lax.dot_general(a, w_ref[...], (((1,), (0,)), ((), ())),
                        preferred_element_type=jnp.float32)
    s_ref[...] += jnp.sum(h, axis=0, keepdims=True)
    q_ref[...] += jnp.sum(h * h, axis=0, keepdims=True)


def _stats_layer(x, w, scale, shift):
    ci, co = w.shape
    grid = (_P // _PB,)
    return pl.pallas_call(
        _stats_body,
        grid=grid,
        in_specs=[
            pl.BlockSpec((_PB, ci), lambda i: (i, 0)),
            pl.BlockSpec((ci, co), lambda i: (0, 0)),
            pl.BlockSpec((1, ci), lambda i: (0, 0)),
            pl.BlockSpec((1, ci), lambda i: (0, 0)),
        ],
        out_specs=[
            pl.BlockSpec((1, co), lambda i: (0, 0)),
            pl.BlockSpec((1, co), lambda i: (0, 0)),
        ],
        out_shape=[
            jax.ShapeDtypeStruct((1, co), jnp.float32),
            jax.ShapeDtypeStruct((1, co), jnp.float32),
        ],
    )(x, w, scale, shift)


def _pool_body(h_ref, w_ref, sc1_ref, sh1_ref, sc2_ref, sh2_ref, o_ref):
    a1 = jnp.maximum(h_ref[...] * sc1_ref[...] + sh1_ref[...], 0.0)
    h2 = lax.dot_general(a1, w_ref[...], (((1,), (0,)), ((), ())),
                         preferred_element_type=jnp.float32)
    a = jnp.maximum(h2 * sc2_ref[...] + sh2_ref[...], 0.0)
    a = a.reshape(_MBF, _K, 256)
    m = jnp.max(a, axis=1)
    o_ref[0] = m.T


def _pool(h1, w2, sc1, sh1, sc2, sh2):
    grid = (_B, _M // _MBF)
    return pl.pallas_call(
        _pool_body,
        grid=grid,
        in_specs=[
            pl.BlockSpec((_MBF * _K, 128),
                         lambda b, m: (b * (_M // _MBF) + m, 0)),
            pl.BlockSpec((128, 256), lambda b, m: (0, 0)),
            pl.BlockSpec((1, 128), lambda b, m: (0, 0)),
            pl.BlockSpec((1, 128), lambda b, m: (0, 0)),
            pl.BlockSpec((1, 256), lambda b, m: (0, 0)),
            pl.BlockSpec((1, 256), lambda b, m: (0, 0)),
        ],
        out_specs=pl.BlockSpec((1, 256, _MBF), lambda b, m: (b, 0, m)),
        out_shape=jax.ShapeDtypeStruct((_B, 256, _M), jnp.float32),
    )(h1, w2, sc1, sh1, sc2, sh2)


def _stats_to_affine(s, q, gamma, beta):
    mean = s[0] / _P
    var = q[0] / _P - mean * mean
    scale = gamma / jnp.sqrt(var + 1e-5)
    shift = beta - mean * scale
    return scale[None, :], shift[None, :]


def kernel(xyz, features, W0, gamma0, beta0, W1, gamma1, beta1, W2, gamma2, beta2):
    perm = jax.random.permutation(jax.random.key(1), _N)[:_M]
    new_xyz = jnp.take(xyz, perm, axis=1)

    nxyz_pad = jnp.pad(new_xyz, ((0, 0), (0, 0), (0, 5))).reshape(_B * _M, 8)
    xyzt_pad = jnp.pad(jnp.transpose(xyz, (0, 2, 1)),
                       ((0, 0), (0, 5), (0, 0)))

    idx = _knn_ids(nxyz_pad, xyzt_pad)
    idx3 = idx.reshape(_NW, _NCH, _CH)

    tab = jnp.concatenate(
        [jnp.transpose(features, (0, 2, 1)),
         jnp.pad(xyz, ((0, 0), (0, 0), (0, 61)))], axis=2).reshape(_B * _N, 128)

    g = _sc_gather(idx3, tab).reshape(_P, 128)

    nxyz16 = jnp.pad(new_xyz, ((0, 0), (0, 0), (0, 13))).reshape(_B * _M, 16)

    w0c = jnp.pad(jnp.transpose(W0), ((0, 61), (0, 0)))
    w0x = jnp.pad(jnp.transpose(W0[:, _C:]), ((0, 13), (0, 0)))

    h0, s0, q0 = _layer0(g, nxyz16, w0c, w0x)
    sc0, sh0 = _stats_to_affine(s0, q0, gamma0, beta0)

    h1, s1, q1 = _mid_layer(h0, jnp.transpose(W1), sc0, sh0)
    sc1, sh1 = _stats_to_affine(s1, q1, gamma1, beta1)

    w2t = jnp.transpose(W2)
    s2, q2 = _stats_layer(h1, w2t, sc1, sh1)
    sc2, sh2 = _stats_to_affine(s2, q2, gamma2, beta2)

    new_features = _pool(h1, w2t, sc1, sh1, sc2, sh2)
    return new_xyz, new_features


# speedup vs baseline: 10.9679x; 1.0746x over previous
import functools

import jax
import jax.numpy as jnp
from jax import lax
from jax.experimental import pallas as pl
from jax.experimental.pallas import tpu as pltpu
from jax.experimental.pallas import tpu_sc as plsc

_B, _N, _C = 8, 4096, 64
_M, _K = 1024, 32
_P = _B * _M * _K

_MB = 512
_PB = 8192
_MBF = 256

_NW = 32
_RW = _P // _NW
_CH = 128
_NCH = _RW // _CH

_BIG = 1e30


def _knn_body(nxyz_ref, xyzt_ref, idx_ref, d_ref, acc_ref):
    b = pl.program_id(0)
    nb = nxyz_ref[...]
    xb = xyzt_ref[0]
    s2 = jnp.sum(nb * nb, axis=1, keepdims=True)
    d2 = jnp.sum(xb * xb, axis=0, keepdims=True)
    cross = lax.dot_general(nb, xb, (((1,), (0,)), ((), ())),
                            preferred_element_type=jnp.float32)
    d0 = s2 + d2 - 2.0 * cross
    d_ref[...] = d0
    m0 = jnp.min(d0, axis=1, keepdims=True)

    lane_n = lax.broadcasted_iota(jnp.int32, (_MB, _N), 1).astype(jnp.float32)
    lane_k = lax.broadcasted_iota(jnp.int32, (_MB, _K), 1).astype(jnp.float32)

    def body(k, m):
        d = d_ref[...]
        am = jnp.min(jnp.where(d <= m, lane_n, jnp.float32(_N)),
                     axis=1, keepdims=True)
        dn = jnp.where(lane_n == am, _BIG, d)
        d_ref[...] = dn
        acc_ref[...] = jnp.where(lane_k == k.astype(jnp.float32),
                                 jnp.broadcast_to(am, (_MB, _K)), acc_ref[...])
        return jnp.min(dn, axis=1, keepdims=True)

    lax.fori_loop(0, _K, body, m0)
    idx_ref[...] = (acc_ref[...] + (b * _N).astype(jnp.float32)).astype(jnp.int32)


def _knn_ids(nxyz_pad, xyzt_pad):
    grid = (_B, _M // _MB)
    return pl.pallas_call(
        _knn_body,
        grid=grid,
        in_specs=[
            pl.BlockSpec((_MB, 8), lambda b, m: (b * (_M // _MB) + m, 0)),
            pl.BlockSpec((1, 8, _N), lambda b, m: (b, 0, 0)),
        ],
        out_specs=pl.BlockSpec((_MB, _K), lambda b, m: (b * (_M // _MB) + m, 0)),
        out_shape=jax.ShapeDtypeStruct((_B * _M, _K), jnp.int32),
        scratch_shapes=[pltpu.VMEM((_MB, _N), jnp.float32),
                        pltpu.VMEM((_MB, _K), jnp.float32)],
    )(nxyz_pad, xyzt_pad)


def _sc_gather(idx3, tab):
    mesh = plsc.VectorSubcoreMesh(core_axis_name="c", subcore_axis_name="s")

    @functools.partial(
        pl.kernel,
        mesh=mesh,
        out_type=jax.ShapeDtypeStruct((_NW, _NCH, _CH, 128), jnp.float32),
        scratch_types=[
            pltpu.VMEM((_NCH, _CH), jnp.int32),
            pltpu.VMEM((_CH, 128), jnp.float32),
            pltpu.VMEM((_CH, 128), jnp.float32),
            pltpu.SemaphoreType.DMA,
            pltpu.SemaphoreType.DMA,
        ],
    )
    def k(idx_hbm, tab_hbm, out_hbm, idx_v, buf0, buf1, sem0, sem1):
        wid = lax.axis_index("s") * 2 + lax.axis_index("c")
        pltpu.sync_copy(idx_hbm.at[wid], idx_v)

        pltpu.async_copy(tab_hbm.at[idx_v.at[0]], buf0, sem0)

        def step(j2, _):
            j = j2 * 2
            pltpu.async_copy(tab_hbm.at[idx_v.at[j + 1]], buf1, sem1)
            pltpu.make_async_copy(tab_hbm.at[idx_v.at[j]], buf0, sem0).wait()
            pltpu.sync_copy(buf0, out_hbm.at[wid, j])

            @pl.when(j + 2 < _NCH)
            def _():
                pltpu.async_copy(tab_hbm.at[idx_v.at[j + 2]], buf0, sem0)

            pltpu.make_async_copy(tab_hbm.at[idx_v.at[j + 1]], buf1, sem1).wait()
            pltpu.sync_copy(buf1, out_hbm.at[wid, j + 1])
            return 0

        lax.fori_loop(0, _NCH // 2, step, 0)

    return k(idx3, tab)


def _l0_body(g_ref, nx_ref, wc_ref, wx_ref, h_ref, s_ref, q_ref):
    @pl.when(pl.program_id(0) == 0)
    def _():
        s_ref[...] = jnp.zeros_like(s_ref)
        q_ref[...] = jnp.zeros_like(q_ref)

    g = g_ref[...]
    nx = nx_ref[...]
    corr = lax.dot_general(nx, wx_ref[...], (((1,), (0,)), ((), ())),
                           preferred_element_type=jnp.float32)
    corrb = jnp.broadcast_to(corr[:, None, :],
                             (_PB // _K, _K, 64)).reshape(_PB, 64)
    h = lax.dot_general(g, wc_ref[...], (((1,), (0,)), ((), ())),
                        preferred_element_type=jnp.float32) - corrb
    h_ref[...] = h
    s_ref[...] += jnp.sum(h, axis=0, keepdims=True)
    q_ref[...] += jnp.sum(h * h, axis=0, keepdims=True)


def _layer0(g, nxyz16, wc, wx):
    grid = (_P // _PB,)
    return pl.pallas_call(
        _l0_body,
        grid=grid,
        in_specs=[
            pl.BlockSpec((_PB, 128), lambda i: (i, 0)),
            pl.BlockSpec((_PB // _K, 16), lambda i: (i, 0)),
            pl.BlockSpec((128, 64), lambda i: (0, 0)),
            pl.BlockSpec((16, 64), lambda i: (0, 0)),
        ],
        out_specs=[
            pl.BlockSpec((_PB, 64), lambda i: (i, 0)),
            pl.BlockSpec((1, 64), lambda i: (0, 0)),
            pl.BlockSpec((1, 64), lambda i: (0, 0)),
        ],
        out_shape=[
            jax.ShapeDtypeStruct((_P, 64), jnp.float32),
            jax.ShapeDtypeStruct((1, 64), jnp.float32),
            jax.ShapeDtypeStruct((1, 64), jnp.float32),
        ],
    )(g, nxyz16, wc, wx)


def _mid_body(x_ref, w_ref, sc_ref, sh_ref, h_ref, s_ref, q_ref):
    @pl.when(pl.program_id(0) == 0)
    def _():
        s_ref[...] = jnp.zeros_like(s_ref)
        q_ref[...] = jnp.zeros_like(q_ref)

    a = jnp.maximum(x_ref[...] * sc_ref[...] + sh_ref[...], 0.0)
    h = lax.dot_general(a, w_ref[...], (((1,), (0,)), ((), ())),
                        preferred_element_type=jnp.float32)
    h_ref[...] = h
    s_ref[...] += jnp.sum(h, axis=0, keepdims=True)
    q_ref[...] += jnp.sum(h * h, axis=0, keepdims=True)


def _mid_layer(x, w, scale, shift):
    ci, co = w.shape
    grid = (_P // _PB,)
    return pl.pallas_call(
        _mid_body,
        grid=grid,
        in_specs=[
            pl.BlockSpec((_PB, ci), lambda i: (i, 0)),
            pl.BlockSpec((ci, co), lambda i: (0, 0)),
            pl.BlockSpec((1, ci), lambda i: (0, 0)),
            pl.BlockSpec((1, ci), lambda i: (0, 0)),
        ],
        out_specs=[
            pl.BlockSpec((_PB, co), lambda i: (i, 0)),
            pl.BlockSpec((1, co), lambda i: (0, 0)),
            pl.BlockSpec((1, co), lambda i: (0, 0)),
        ],
        out_shape=[
            jax.ShapeDtypeStruct((_P, co), jnp.float32),
            jax.ShapeDtypeStruct((1, co), jnp.float32),
            jax.ShapeDtypeStruct((1, co), jnp.float32),
        ],
    )(x, w, scale, shift)


def _stats_body(x_ref, w_ref, sc_ref, sh_ref, s_ref, q_ref):
    @pl.when(pl.program_id(0) == 0)
    def _():
        s_ref[...] = jnp.zeros_like(s_ref)
        q_ref[...] = jnp.zeros_like(q_ref)

    a = jnp.maximum(x_ref[...] * sc_ref[...] + sh_ref[...], 0.0)
    h = lax.dot_general(a, w_ref[...], (((1,), (0,)), ((), ())),
                        preferred_element_type=jnp.float32)
    s_ref[...] += jnp.sum(h, axis=0, keepdims=True)
    q_ref[...] += jnp.sum(h * h, axis=0, keepdims=True)


def _stats_layer(x, w, scale, shift):
    ci, co = w.shape
    grid = (_P // _PB,)
    return pl.pallas_call(
        _stats_body,
        grid=grid,
        in_specs=[
            pl.BlockSpec((_PB, ci), lambda i: (i, 0)),
            pl.BlockSpec((ci, co), lambda i: (0, 0)),
            pl.BlockSpec((1, ci), lambda i: (0, 0)),
            pl.BlockSpec((1, ci), lambda i: (0, 0)),
        ],
        out_specs=[
            pl.BlockSpec((1, co), lambda i: (0, 0)),
            pl.BlockSpec((1, co), lambda i: (0, 0)),
        ],
        out_shape=[
            jax.ShapeDtypeStruct((1, co), jnp.float32),
            jax.ShapeDtypeStruct((1, co), jnp.float32),
        ],
    )(x, w, scale, shift)


def _pool_body(h_ref, w_ref, sc1_ref, sh1_ref, sc2_ref, sh2_ref, o_ref):
    a1 = jnp.maximum(h_ref[...] * sc1_ref[...] + sh1_ref[...], 0.0)
    h2 = lax.dot_general(a1, w_ref[...], (((1,), (0,)), ((), ())),
                         preferred_element_type=jnp.float32)
    a = jnp.maximum(h2 * sc2_ref[...] + sh2_ref[...], 0.0)
    a = a.reshape(_MBF, _K, 256)
    m = jnp.max(a, axis=1)
    o_ref[0] = m.T


def _pool(h1, w2, sc1, sh1, sc2, sh2):
    grid = (_B, _M // _MBF)
    return pl.pallas_call(
        _pool_body,
        grid=grid,
        in_specs=[
            pl.BlockSpec((_MBF * _K, 128),
                         lambda b, m: (b * (_M // _MBF) + m, 0)),
            pl.BlockSpec((128, 256), lambda b, m: (0, 0)),
            pl.BlockSpec((1, 128), lambda b, m: (0, 0)),
            pl.BlockSpec((1, 128), lambda b, m: (0, 0)),
            pl.BlockSpec((1, 256), lambda b, m: (0, 0)),
            pl.BlockSpec((1, 256), lambda b, m: (0, 0)),
        ],
        out_specs=pl.BlockSpec((1, 256, _MBF), lambda b, m: (b, 0, m)),
        out_shape=jax.ShapeDtypeStruct((_B, 256, _M), jnp.float32),
    )(h1, w2, sc1, sh1, sc2, sh2)


def _stats_to_affine(s, q, gamma, beta):
    mean = s[0] / _P
    var = q[0] / _P - mean * mean
    scale = gamma / jnp.sqrt(var + 1e-5)
    shift = beta - mean * scale
    return scale[None, :], shift[None, :]


def kernel(xyz, features, W0, gamma0, beta0, W1, gamma1, beta1, W2, gamma2, beta2):
    perm = jax.random.permutation(jax.random.key(1), _N)[:_M]
    new_xyz = jnp.take(xyz, perm, axis=1)

    nxyz_pad = jnp.pad(new_xyz, ((0, 0), (0, 0), (0, 5))).reshape(_B * _M, 8)
    xyzt_pad = jnp.pad(jnp.transpose(xyz, (0, 2, 1)),
                       ((0, 0), (0, 5), (0, 0)))

    idx = _knn_ids(nxyz_pad, xyzt_pad)
    idx3 = idx.reshape(_NW, _NCH, _CH)

    tab = jnp.concatenate(
        [jnp.transpose(features, (0, 2, 1)),
         jnp.pad(xyz, ((0, 0), (0, 0), (0, 61)))], axis=2).reshape(_B * _N, 128)

    g = _sc_gather(idx3, tab).reshape(_P, 128)

    nxyz16 = jnp.pad(new_xyz, ((0, 0), (0, 0), (0, 13))).reshape(_B * _M, 16)

    w0c = jnp.pad(jnp.transpose(W0), ((0, 61), (0, 0)))
    w0x = jnp.pad(jnp.transpose(W0[:, _C:]), ((0, 13), (0, 0)))

    h0, s0, q0 = _layer0(g, nxyz16, w0c, w0x)
    sc0, sh0 = _stats_to_affine(s0, q0, gamma0, beta0)

    h1, s1, q1 = _mid_layer(h0, jnp.transpose(W1), sc0, sh0)
    sc1, sh1 = _stats_to_affine(s1, q1, gamma1, beta1)

    w2t = jnp.transpose(W2)
    s2, q2 = _stats_layer(h1, w2t, sc1, sh1)
    sc2, sh2 = _stats_to_affine(s2, q2, gamma2, beta2)

    new_features = _pool(h1, w2t, sc1, sh1, sc2, sh2)
    return new_xyz, new_features


# speedup vs baseline: 11.2936x; 1.0297x over previous
import functools

import jax
import jax.numpy as jnp
from jax import lax
from jax.experimental import pallas as pl
from jax.experimental.pallas import tpu as pltpu
from jax.experimental.pallas import tpu_sc as plsc

_B, _N, _C = 8, 4096, 64
_M, _K = 1024, 32
_P = _B * _M * _K

_MB = 1024
_PB = 16384
_MBF = 512

_NW = 32
_RW = _P // _NW
_CH = 128
_NCH = _RW // _CH

_BIG = 1e30


def _knn_body(nxyz_ref, xyzt_ref, idx_ref, d_ref, acc_ref):
    b = pl.program_id(0)
    nb = nxyz_ref[...]
    xb = xyzt_ref[0]
    s2 = jnp.sum(nb * nb, axis=1, keepdims=True)
    d2 = jnp.sum(xb * xb, axis=0, keepdims=True)
    cross = lax.dot_general(nb, xb, (((1,), (0,)), ((), ())),
                            preferred_element_type=jnp.float32)
    d0 = s2 + d2 - 2.0 * cross
    d_ref[...] = d0
    m0 = jnp.min(d0, axis=1, keepdims=True)

    lane_n = lax.broadcasted_iota(jnp.int32, (_MB, _N), 1).astype(jnp.float32)
    lane_k = lax.broadcasted_iota(jnp.int32, (_MB, _K), 1).astype(jnp.float32)

    def body(k, m):
        d = d_ref[...]
        am = jnp.min(jnp.where(d <= m, lane_n, jnp.float32(_N)),
                     axis=1, keepdims=True)
        dn = jnp.where(lane_n == am, _BIG, d)
        d_ref[...] = dn
        acc_ref[...] = jnp.where(lane_k == k.astype(jnp.float32),
                                 jnp.broadcast_to(am, (_MB, _K)), acc_ref[...])
        return jnp.min(dn, axis=1, keepdims=True)

    lax.fori_loop(0, _K, body, m0)
    idx_ref[...] = (acc_ref[...] + (b * _N).astype(jnp.float32)).astype(jnp.int32)


def _knn_ids(nxyz_pad, xyzt_pad):
    grid = (_B, _M // _MB)
    return pl.pallas_call(
        _knn_body,
        grid=grid,
        in_specs=[
            pl.BlockSpec((_MB, 8), lambda b, m: (b * (_M // _MB) + m, 0)),
            pl.BlockSpec((1, 8, _N), lambda b, m: (b, 0, 0)),
        ],
        out_specs=pl.BlockSpec((_MB, _K), lambda b, m: (b * (_M // _MB) + m, 0)),
        out_shape=jax.ShapeDtypeStruct((_B * _M, _K), jnp.int32),
        scratch_shapes=[pltpu.VMEM((_MB, _N), jnp.float32),
                        pltpu.VMEM((_MB, _K), jnp.float32)],
    )(nxyz_pad, xyzt_pad)


def _sc_gather(idx3, tab):
    mesh = plsc.VectorSubcoreMesh(core_axis_name="c", subcore_axis_name="s")

    @functools.partial(
        pl.kernel,
        mesh=mesh,
        out_type=jax.ShapeDtypeStruct((_NW, _NCH, _CH, 128), jnp.float32),
        scratch_types=[
            pltpu.VMEM((_NCH, _CH), jnp.int32),
            pltpu.VMEM((_CH, 128), jnp.float32),
            pltpu.VMEM((_CH, 128), jnp.float32),
            pltpu.SemaphoreType.DMA,
            pltpu.SemaphoreType.DMA,
        ],
    )
    def k(idx_hbm, tab_hbm, out_hbm, idx_v, buf0, buf1, sem0, sem1):
        wid = lax.axis_index("s") * 2 + lax.axis_index("c")
        pltpu.sync_copy(idx_hbm.at[wid], idx_v)

        pltpu.async_copy(tab_hbm.at[idx_v.at[0]], buf0, sem0)

        def step(j2, _):
            j = j2 * 2
            pltpu.async_copy(tab_hbm.at[idx_v.at[j + 1]], buf1, sem1)
            pltpu.make_async_copy(tab_hbm.at[idx_v.at[j]], buf0, sem0).wait()
            pltpu.sync_copy(buf0, out_hbm.at[wid, j])

            @pl.when(j + 2 < _NCH)
            def _():
                pltpu.async_copy(tab_hbm.at[idx_v.at[j + 2]], buf0, sem0)

            pltpu.make_async_copy(tab_hbm.at[idx_v.at[j + 1]], buf1, sem1).wait()
            pltpu.sync_copy(buf1, out_hbm.at[wid, j + 1])
            return 0

        lax.fori_loop(0, _NCH // 2, step, 0)

    return k(idx3, tab)


def _l0_body(g_ref, nx_ref, wc_ref, wx_ref, h_ref, s_ref, q_ref):
    @pl.when(pl.program_id(0) == 0)
    def _():
        s_ref[...] = jnp.zeros_like(s_ref)
        q_ref[...] = jnp.zeros_like(q_ref)

    g = g_ref[...]
    nx = nx_ref[...]
    corr = lax.dot_general(nx, wx_ref[...], (((1,), (0,)), ((), ())),
                           preferred_element_type=jnp.float32)
    corrb = jnp.broadcast_to(corr[:, None, :],
                             (_PB // _K, _K, 64)).reshape(_PB, 64)
    h = lax.dot_general(g, wc_ref[...], (((1,), (0,)), ((), ())),
                        preferred_element_type=jnp.float32) - corrb
    h_ref[...] = h
    s_ref[...] += jnp.sum(h, axis=0, keepdims=True)
    q_ref[...] += jnp.sum(h * h, axis=0, keepdims=True)


def _layer0(g, nxyz16, wc, wx):
    grid = (_P // _PB,)
    return pl.pallas_call(
        _l0_body,
        grid=grid,
        in_specs=[
            pl.BlockSpec((_PB, 128), lambda i: (i, 0)),
            pl.BlockSpec((_PB // _K, 16), lambda i: (i, 0)),
            pl.BlockSpec((128, 64), lambda i: (0, 0)),
            pl.BlockSpec((16, 64), lambda i: (0, 0)),
        ],
        out_specs=[
            pl.BlockSpec((_PB, 64), lambda i: (i, 0)),
            pl.BlockSpec((1, 64), lambda i: (0, 0)),
            pl.BlockSpec((1, 64), lambda i: (0, 0)),
        ],
        out_shape=[
            jax.ShapeDtypeStruct((_P, 64), jnp.float32),
            jax.ShapeDtypeStruct((1, 64), jnp.float32),
            jax.ShapeDtypeStruct((1, 64), jnp.float32),
        ],
    )(g, nxyz16, wc, wx)


def _mid_body(x_ref, w_ref, sc_ref, sh_ref, h_ref, s_ref, q_ref):
    @pl.when(pl.program_id(0) == 0)
    def _():
        s_ref[...] = jnp.zeros_like(s_ref)
        q_ref[...] = jnp.zeros_like(q_ref)

    a = jnp.maximum(x_ref[...] * sc_ref[...] + sh_ref[...], 0.0)
    h = lax.dot_general(a, w_ref[...], (((1,), (0,)), ((), ())),
                        preferred_element_type=jnp.float32)
    h_ref[...] = h
    s_ref[...] += jnp.sum(h, axis=0, keepdims=True)
    q_ref[...] += jnp.sum(h * h, axis=0, keepdims=True)


def _mid_layer(x, w, scale, shift):
    ci, co = w.shape
    grid = (_P // _PB,)
    return pl.pallas_call(
        _mid_body,
        grid=grid,
        in_specs=[
            pl.BlockSpec((_PB, ci), lambda i: (i, 0)),
            pl.BlockSpec((ci, co), lambda i: (0, 0)),
            pl.BlockSpec((1, ci), lambda i: (0, 0)),
            pl.BlockSpec((1, ci), lambda i: (0, 0)),
        ],
        out_specs=[
            pl.BlockSpec((_PB, co), lambda i: (i, 0)),
            pl.BlockSpec((1, co), lambda i: (0, 0)),
            pl.BlockSpec((1, co), lambda i: (0, 0)),
        ],
        out_shape=[
            jax.ShapeDtypeStruct((_P, co), jnp.float32),
            jax.ShapeDtypeStruct((1, co), jnp.float32),
            jax.ShapeDtypeStruct((1, co), jnp.float32),
        ],
    )(x, w, scale, shift)


def _stats_body(x_ref, w_ref, sc_ref, sh_ref, s_ref, q_ref):
    @pl.when(pl.program_id(0) == 0)
    def _():
        s_ref[...] = jnp.zeros_like(s_ref)
        q_ref[...] = jnp.zeros_like(q_ref)

    a = jnp.maximum(x_ref[...] * sc_ref[...] + sh_ref[...], 0.0)
    h = lax.dot_general(a, w_ref[...], (((1,), (0,)), ((), ())),
                        preferred_element_type=jnp.float32)
    s_ref[...] += jnp.sum(h, axis=0, keepdims=True)
    q_ref[...] += jnp.sum(h * h, axis=0, keepdims=True)


def _stats_layer(x, w, scale, shift):
    ci, co = w.shape
    grid = (_P // _PB,)
    return pl.pallas_call(
        _stats_body,
        grid=grid,
        in_specs=[
            pl.BlockSpec((_PB, ci), lambda i: (i, 0)),
            pl.BlockSpec((ci, co), lambda i: (0, 0)),
            pl.BlockSpec((1, ci), lambda i: (0, 0)),
            pl.BlockSpec((1, ci), lambda i: (0, 0)),
        ],
        out_specs=[
            pl.BlockSpec((1, co), lambda i: (0, 0)),
            pl.BlockSpec((1, co), lambda i: (0, 0)),
        ],
        out_shape=[
            jax.ShapeDtypeStruct((1, co), jnp.float32),
            jax.ShapeDtypeStruct((1, co), jnp.float32),
        ],
    )(x, w, scale, shift)


def _pool_body(h_ref, w_ref, sc1_ref, sh1_ref, sc2_ref, sh2_ref, o_ref):
    a1 = jnp.maximum(h_ref[...] * sc1_ref[...] + sh1_ref[...], 0.0)
    h2 = lax.dot_general(a1, w_ref[...], (((1,), (0,)), ((), ())),
                         preferred_element_type=jnp.float32)
    a = jnp.maximum(h2 * sc2_ref[...] + sh2_ref[...], 0.0)
    a = a.reshape(_MBF, _K, 256)
    m = jnp.max(a, axis=1)
    o_ref[0] = m.T


def _pool(h1, w2, sc1, sh1, sc2, sh2):
    grid = (_B, _M // _MBF)
    return pl.pallas_call(
        _pool_body,
        grid=grid,
        in_specs=[
            pl.BlockSpec((_MBF * _K, 128),
                         lambda b, m: (b * (_M // _MBF) + m, 0)),
            pl.BlockSpec((128, 256), lambda b, m: (0, 0)),
            pl.BlockSpec((1, 128), lambda b, m: (0, 0)),
            pl.BlockSpec((1, 128), lambda b, m: (0, 0)),
            pl.BlockSpec((1, 256), lambda b, m: (0, 0)),
            pl.BlockSpec((1, 256), lambda b, m: (0, 0)),
        ],
        out_specs=pl.BlockSpec((1, 256, _MBF), lambda b, m: (b, 0, m)),
        out_shape=jax.ShapeDtypeStruct((_B, 256, _M), jnp.float32),
    )(h1, w2, sc1, sh1, sc2, sh2)


def _stats_to_affine(s, q, gamma, beta):
    mean = s[0] / _P
    var = q[0] / _P - mean * mean
    scale = gamma / jnp.sqrt(var + 1e-5)
    shift = beta - mean * scale
    return scale[None, :], shift[None, :]


def kernel(xyz, features, W0, gamma0, beta0, W1, gamma1, beta1, W2, gamma2, beta2):
    perm = jax.random.permutation(jax.random.key(1), _N)[:_M]
    new_xyz = jnp.take(xyz, perm, axis=1)

    nxyz_pad = jnp.pad(new_xyz, ((0, 0), (0, 0), (0, 5))).reshape(_B * _M, 8)
    xyzt_pad = jnp.pad(jnp.transpose(xyz, (0, 2, 1)),
                       ((0, 0), (0, 5), (0, 0)))

    idx = _knn_ids(nxyz_pad, xyzt_pad)
    idx3 = idx.reshape(_NW, _NCH, _CH)

    tab = jnp.concatenate(
        [jnp.transpose(features, (0, 2, 1)),
         jnp.pad(xyz, ((0, 0), (0, 0), (0, 61)))], axis=2).reshape(_B * _N, 128)

    g = _sc_gather(idx3, tab).reshape(_P, 128)

    nxyz16 = jnp.pad(new_xyz, ((0, 0), (0, 0), (0, 13))).reshape(_B * _M, 16)

    w0c = jnp.pad(jnp.transpose(W0), ((0, 61), (0, 0)))
    w0x = jnp.pad(jnp.transpose(W0[:, _C:]), ((0, 13), (0, 0)))

    h0, s0, q0 = _layer0(g, nxyz16, w0c, w0x)
    sc0, sh0 = _stats_to_affine(s0, q0, gamma0, beta0)

    h1, s1, q1 = _mid_layer(h0, jnp.transpose(W1), sc0, sh0)
    sc1, sh1 = _stats_to_affine(s1, q1, gamma1, beta1)

    w2t = jnp.transpose(W2)
    s2, q2 = _stats_layer(h1, w2t, sc1, sh1)
    sc2, sh2 = _stats_to_affine(s2, q2, gamma2, beta2)

    new_features = _pool(h1, w2t, sc1, sh1, sc2, sh2)
    return new_xyz, new_features
